# Initial kernel scaffold; baseline (speedup 1.0000x reference)
import jax
import jax.numpy as jnp
from jax.experimental import pallas as pl


def kernel(x, edge_index, W, b):
    raise NotImplementedError("write your pallas kernel here")


# speedup vs baseline: 39.9717x; 39.9717x over previous
import functools

import jax
import jax.numpy as jnp
from jax import lax
from jax.experimental import pallas as pl
from jax.experimental.pallas import tpu as pltpu
from jax.experimental.pallas import tpu_sc as plsc

N = 10000
D = 128
C = 10
E = 320000

L = 16
NPAD = 10240
NSC = 2
NTILE = 16
NW = NSC * NTILE
SL = NPAD // NTILE
CHUNK = 128
EPT = 10240
NCH = EPT // CHUNK
EPAD = NW * EPT

@functools.cache
def _mesh():
    return plsc.VectorSubcoreMesh(core_axis_name="c", subcore_axis_name="s",
                                  num_cores=NSC, num_subcores=NTILE)


def _deg_body(dst_hbm, z1_hbm, degp_hbm, acc_v, idx_v, tmp_v, red_v, shacc):
    cid = lax.axis_index("c")
    sid = lax.axis_index("s")
    wid = cid * NTILE + sid
    pltpu.sync_copy(z1_hbm, acc_v)
    pltpu.sync_copy(dst_hbm.at[pl.ds(wid * EPT, EPT)], idx_v)
    ones = jnp.ones((L,), jnp.float32)

    def scat(j, carry):
        idx = idx_v[pl.ds(j * L, L)]
        plsc.addupdate_scatter(acc_v, [idx], ones)
        return carry

    lax.fori_loop(0, EPT // L, scat, 0)

    pltpu.sync_copy(acc_v, shacc.at[sid])
    plsc.subcore_barrier()
    for r in range(NTILE):
        pltpu.sync_copy(shacc.at[r, pl.ds(sid * SL, SL)], tmp_v.at[r])

    def red(c, carry):
        s = jnp.zeros((L,), jnp.float32)
        for r in range(NTILE):
            s = s + tmp_v[r, pl.ds(c * L, L)]
        red_v[pl.ds(c * L, L)] = s
        return carry

    lax.fori_loop(0, SL // L, red, 0)
    pltpu.sync_copy(red_v, degp_hbm.at[cid, pl.ds(sid * SL, SL)])


@functools.cache
def _deg():
    return pl.kernel(
        _deg_body,
        out_type=jax.ShapeDtypeStruct((NSC, NPAD), jnp.float32),
        mesh=_mesh(),
        compiler_params=pltpu.CompilerParams(needs_layout_passes=False),
        scratch_types=[
            pltpu.VMEM((NPAD,), jnp.float32),
            pltpu.VMEM((EPT,), jnp.int32),
            pltpu.VMEM((NTILE, SL), jnp.float32),
            pltpu.VMEM((SL,), jnp.float32),
            pltpu.VMEM_SHARED((NTILE, NPAD), jnp.float32),
        ],
    )


def _hop_body(h_hbm, srcs_hbm, dsts_hbm, z2_hbm, p_hbm,
              idxs_v, idxd_v, rows_v, acc_s, sem):
    cid = lax.axis_index("c")
    sid = lax.axis_index("s")
    wid = cid * NTILE + sid
    off = sid * SL
    pltpu.sync_copy(z2_hbm, acc_s.at[pl.ds(off, SL)])
    pltpu.sync_copy(srcs_hbm.at[wid], idxs_v)
    pltpu.sync_copy(dsts_hbm.at[wid], idxd_v)
    plsc.subcore_barrier()

    def gather(c, b):
        return pltpu.make_async_copy(h_hbm.at[idxs_v.at[c]], rows_v.at[b],
                                     sem.at[b])

    gather(0, 0).start()

    def step(k, carry):
        c0 = 2 * k
        gather(c0 + 1, 1).start()
        gather(c0, 0).wait()
        pltpu.sync_copy(rows_v.at[0], acc_s.at[idxd_v.at[c0]], add=True)

        @pl.when(k < NCH // 2 - 1)
        def _():
            gather(c0 + 2, 0).start()

        gather(c0 + 1, 1).wait()
        pltpu.sync_copy(rows_v.at[1], acc_s.at[idxd_v.at[c0 + 1]], add=True)
        return carry

    lax.fori_loop(0, NCH // 2, step, 0)
    plsc.subcore_barrier()
    pltpu.sync_copy(acc_s.at[pl.ds(off, SL)], p_hbm.at[cid, pl.ds(off, SL)])


@functools.cache
def _hop():
    return pl.kernel(
        _hop_body,
        out_type=jax.ShapeDtypeStruct((NSC, NPAD, L), jnp.float32),
        mesh=_mesh(),
        compiler_params=pltpu.CompilerParams(needs_layout_passes=False,
                                             use_tc_tiling_on_sc=False),
        scratch_types=[
            pltpu.VMEM((NCH, CHUNK), jnp.int32),
            pltpu.VMEM((NCH, CHUNK), jnp.int32),
            pltpu.VMEM((2, CHUNK, L), jnp.float32),
            pltpu.VMEM_SHARED((NPAD, L), jnp.float32),
            pltpu.SemaphoreType.DMA((2,)),
        ],
    )


_BN = 1024


def _mm_body(x_ref, w_ref, o_ref):
    o_ref[...] = jnp.dot(x_ref[...], w_ref[...],
                         preferred_element_type=jnp.float32)


def _prep_body(degp_ref, g_ref, h0_ref, dis_ref, dinv_ref):
    deg = degp_ref[0] + degp_ref[1] + 1.0
    dis = lax.rsqrt(deg)
    dinv = 1.0 / deg
    h0_ref[...] = g_ref[...] * dis
    dis_ref[...] = dis
    dinv_ref[...] = dinv


def _comb_body(p_ref, h_ref, s_ref, o_ref):
    o_ref[...] = (p_ref[0] + p_ref[1] + h_ref[...]) * s_ref[...]


def _final_body(q_ref, h_ref, s_ref, b_ref, o_ref):
    o_ref[...] = (q_ref[0] + q_ref[1] + h_ref[...]) * s_ref[...] + b_ref[...]


def _mm(x, wp):
    return pl.pallas_call(
        _mm_body,
        grid=(NPAD // _BN,),
        in_specs=[pl.BlockSpec((_BN, D), lambda i: (i, 0)),
                  pl.BlockSpec((D, L), lambda i: (0, 0))],
        out_specs=pl.BlockSpec((_BN, L), lambda i: (i, 0)),
        out_shape=jax.ShapeDtypeStruct((NPAD, L), jnp.float32),
    )(x, wp)


def _prep(degp3, g):
    return pl.pallas_call(
        _prep_body,
        grid=(NPAD // _BN,),
        in_specs=[pl.BlockSpec((NSC, _BN, 1), lambda i: (0, i, 0)),
                  pl.BlockSpec((_BN, L), lambda i: (i, 0))],
        out_specs=[pl.BlockSpec((_BN, L), lambda i: (i, 0)),
                   pl.BlockSpec((_BN, 1), lambda i: (i, 0)),
                   pl.BlockSpec((_BN, 1), lambda i: (i, 0))],
        out_shape=[jax.ShapeDtypeStruct((NPAD, L), jnp.float32),
                   jax.ShapeDtypeStruct((NPAD, 1), jnp.float32),
                   jax.ShapeDtypeStruct((NPAD, 1), jnp.float32)],
    )(degp3, g)


def _comb(p, h, s):
    return pl.pallas_call(
        _comb_body,
        grid=(NPAD // _BN,),
        in_specs=[pl.BlockSpec((NSC, _BN, L), lambda i: (0, i, 0)),
                  pl.BlockSpec((_BN, L), lambda i: (i, 0)),
                  pl.BlockSpec((_BN, 1), lambda i: (i, 0))],
        out_specs=pl.BlockSpec((_BN, L), lambda i: (i, 0)),
        out_shape=jax.ShapeDtypeStruct((NPAD, L), jnp.float32),
    )(p, h, s)


def _final(q, h, s, b16):
    return pl.pallas_call(
        _final_body,
        grid=(NPAD // _BN,),
        in_specs=[pl.BlockSpec((NSC, _BN, L), lambda i: (0, i, 0)),
                  pl.BlockSpec((_BN, L), lambda i: (i, 0)),
                  pl.BlockSpec((_BN, 1), lambda i: (i, 0)),
                  pl.BlockSpec((1, L), lambda i: (0, 0))],
        out_specs=pl.BlockSpec((_BN, L), lambda i: (i, 0)),
        out_shape=jax.ShapeDtypeStruct((NPAD, L), jnp.float32),
    )(q, h, s, b16)


def kernel(x, edge_index, W, b):
    src = edge_index[0]
    dst = edge_index[1]
    padi = jnp.full((EPAD - E,), N, jnp.int32)
    srcs = jnp.concatenate([src, padi]).reshape(NW, NCH, CHUNK)
    dst_flat = jnp.concatenate([dst, padi])
    dsts = dst_flat.reshape(NW, NCH, CHUNK)

    wp = jnp.pad(W, ((0, 0), (0, L - C)))
    b16 = jnp.pad(b, (0, L - C)).reshape(1, L)
    z1 = jnp.zeros((NPAD,), jnp.float32)
    z2 = jnp.zeros((SL, L), jnp.float32)

    degp = _deg()(dst_flat, z1)
    g = _mm(x, wp)
    h0, dis, dinv = _prep(degp.reshape(NSC, NPAD, 1), g)
    p = _hop()(h0, srcs, dsts, z2)
    h1 = _comb(p, h0, dinv)
    q = _hop()(h1, srcs, dsts, z2)
    out = _final(q, h1, dis, b16)
    return out[:N, :C]


# speedup vs baseline: 42.4067x; 1.0609x over previous
import functools

import jax
import jax.numpy as jnp
from jax import lax
from jax.experimental import pallas as pl
from jax.experimental.pallas import tpu as pltpu
from jax.experimental.pallas import tpu_sc as plsc

N = 10000
D = 128
C = 10
E = 320000

L = 16
NPAD = 10240
NSC = 2
NTILE = 16
NW = NSC * NTILE
SL = NPAD // NTILE
CHUNK = 128
EPT = 10240
NCH = EPT // CHUNK
EPAD = NW * EPT
EDT = E // NW
NB = 8


@functools.cache
def _mesh():
    return plsc.VectorSubcoreMesh(core_axis_name="c", subcore_axis_name="s",
                                  num_cores=NSC, num_subcores=NTILE)


def _deg_body(ei_hbm, z1_hbm, degp_hbm, acc_v, idx_v, tmp_v, red_v, rep_v,
              shacc):
    cid = lax.axis_index("c")
    sid = lax.axis_index("s")
    wid = cid * NTILE + sid
    pltpu.sync_copy(z1_hbm, acc_v)
    pltpu.sync_copy(ei_hbm.at[1, pl.ds(wid * EDT, EDT)], idx_v)

    def scat(j, carry):
        for u in range(5):
            idx = idx_v[pl.ds((j * 5 + u) * L, L)]
            cnt, last = plsc.scan_count(idx)
            plsc.addupdate_scatter(acc_v, [idx], cnt.astype(jnp.float32),
                                   mask=last)
        return carry

    lax.fori_loop(0, EDT // L // 5, scat, 0)

    pltpu.sync_copy(acc_v, shacc.at[sid])
    plsc.subcore_barrier()
    for r in range(NTILE):
        pltpu.sync_copy(shacc.at[r, pl.ds(sid * SL, SL)], tmp_v.at[r])

    def red(c, carry):
        s = jnp.zeros((L,), jnp.float32)
        for r in range(NTILE):
            s = s + tmp_v[r, pl.ds(c * L, L)]
        red_v[pl.ds(c * L, L)] = s
        return carry

    lax.fori_loop(0, SL // L, red, 0)

    def rep(c, carry):
        v16 = red_v[pl.ds(c * L, L)]
        for j in range(L):
            rep_v[c * L + j, :] = jnp.full((L,), v16[j], jnp.float32)
        return carry

    lax.fori_loop(0, SL // L, rep, 0)
    pltpu.sync_copy(rep_v, degp_hbm.at[cid, pl.ds(sid * SL, SL)])


@functools.cache
def _deg():
    return pl.kernel(
        _deg_body,
        out_type=jax.ShapeDtypeStruct((NSC, NPAD, L), jnp.float32),
        mesh=_mesh(),
        compiler_params=pltpu.CompilerParams(needs_layout_passes=False,
                                             use_tc_tiling_on_sc=False),
        scratch_types=[
            pltpu.VMEM((NPAD,), jnp.float32),
            pltpu.VMEM((EDT,), jnp.int32),
            pltpu.VMEM((NTILE, SL), jnp.float32),
            pltpu.VMEM((SL,), jnp.float32),
            pltpu.VMEM((SL, L), jnp.float32),
            pltpu.VMEM_SHARED((NTILE, NPAD), jnp.float32),
        ],
    )


def _hop_body(h_hbm, srcs_hbm, dsts_hbm, z2_hbm, p_hbm,
              idxs_v, idxd_v, rows_v, acc_s, gsem, ssem):
    cid = lax.axis_index("c")
    sid = lax.axis_index("s")
    wid = cid * NTILE + sid
    off = sid * SL
    pltpu.sync_copy(z2_hbm, acc_s.at[pl.ds(off, SL)])
    pltpu.sync_copy(srcs_hbm.at[wid], idxs_v)
    pltpu.sync_copy(dsts_hbm.at[wid], idxd_v)
    plsc.subcore_barrier()

    def g_copy(c, b):
        return pltpu.make_async_copy(h_hbm.at[idxs_v.at[c]], rows_v.at[b],
                                     gsem.at[b])

    def s_wait(c, b):
        pltpu.make_async_copy(rows_v.at[b], acc_s.at[idxd_v.at[c]],
                              ssem.at[b]).wait()

    def tick(t, lo, hi):
        if lo:
            s_wait(t - 4, (t + 4) % NB)
        if hi:
            g_copy(t + 4, (t + 4) % NB).start()
        g_copy(t, t % NB).wait()
        pltpu.async_copy(rows_v.at[t % NB], acc_s.at[idxd_v.at[t]],
                         ssem.at[t % NB], add=True)

    for t in range(4):
        g_copy(t, t).start()
    for t in range(NB):
        tick(t, t >= 4, True)

    def step(k, carry):
        for b in range(NB):
            t = k * NB + b
            s_wait(t - 4, (b + 4) % NB)
            g_copy(t + 4, (b + 4) % NB).start()
            g_copy(t, b).wait()
            pltpu.async_copy(rows_v.at[b], acc_s.at[idxd_v.at[t]],
                             ssem.at[b], add=True)
        return carry

    lax.fori_loop(1, NCH // NB - 1, step, 0)
    for t in range(NCH - NB, NCH):
        tick(t, True, t + 4 < NCH)
    for c in range(NCH - 4, NCH):
        s_wait(c, c % NB)

    plsc.subcore_barrier()
    pltpu.sync_copy(acc_s.at[pl.ds(off, SL)], p_hbm.at[cid, pl.ds(off, SL)])


@functools.cache
def _hop():
    return pl.kernel(
        _hop_body,
        out_type=jax.ShapeDtypeStruct((NSC, NPAD, L), jnp.float32),
        mesh=_mesh(),
        compiler_params=pltpu.CompilerParams(needs_layout_passes=False,
                                             use_tc_tiling_on_sc=False),
        scratch_types=[
            pltpu.VMEM((NCH, CHUNK), jnp.int32),
            pltpu.VMEM((NCH, CHUNK), jnp.int32),
            pltpu.VMEM((NB, CHUNK, L), jnp.float32),
            pltpu.VMEM_SHARED((NPAD, L), jnp.float32),
            pltpu.SemaphoreType.DMA((NB,)),
            pltpu.SemaphoreType.DMA((NB,)),
        ],
    )


def _mm_body(x_ref, w_ref, o_ref):
    o_ref[pl.ds(0, N)] = jnp.dot(x_ref[...], w_ref[...],
                                 preferred_element_type=jnp.float32)
    o_ref[pl.ds(N, NPAD - N)] = jnp.zeros((NPAD - N, L), jnp.float32)


def _prep_body(degp_ref, g_ref, h0_ref, dis_ref, dinv_ref):
    deg = degp_ref[0] + degp_ref[1] + 1.0
    dis = lax.rsqrt(deg)
    dinv = 1.0 / deg
    h0_ref[...] = g_ref[...] * dis
    dis_ref[...] = dis
    dinv_ref[...] = dinv


def _comb_body(p_ref, h_ref, s_ref, o_ref):
    o_ref[...] = (p_ref[0] + p_ref[1] + h_ref[...]) * s_ref[...]


def _final_body(q_ref, h_ref, s_ref, b_ref, o_ref):
    o_ref[...] = (q_ref[0] + q_ref[1] + h_ref[...]) * s_ref[...] + b_ref[...]


def _mm(x, wp):
    return pl.pallas_call(
        _mm_body,
        out_shape=jax.ShapeDtypeStruct((NPAD, L), jnp.float32),
    )(x, wp)


def _prep(degp, g):
    return pl.pallas_call(
        _prep_body,
        out_shape=[jax.ShapeDtypeStruct((NPAD, L), jnp.float32)] * 3,
    )(degp, g)


def _comb(p, h, s):
    return pl.pallas_call(
        _comb_body,
        out_shape=jax.ShapeDtypeStruct((NPAD, L), jnp.float32),
    )(p, h, s)


def _final(q, h, s, b16):
    return pl.pallas_call(
        _final_body,
        out_shape=jax.ShapeDtypeStruct((NPAD, L), jnp.float32),
    )(q, h, s, b16)


def kernel(x, edge_index, W, b):
    src = edge_index[0]
    dst = edge_index[1]
    padi = jnp.full((EPAD - E,), N, jnp.int32)
    srcs = jnp.concatenate([src, padi]).reshape(NW, NCH, CHUNK)
    dsts = jnp.concatenate([dst, padi]).reshape(NW, NCH, CHUNK)

    wp = jnp.pad(W, ((0, 0), (0, L - C)))
    b16 = jnp.pad(b, (0, L - C)).reshape(1, L)
    z1 = jnp.zeros((NPAD,), jnp.float32)
    z2 = jnp.zeros((SL, L), jnp.float32)

    degp = _deg()(edge_index, z1)
    g = _mm(x, wp)
    h0, dis, dinv = _prep(degp, g)
    p = _hop()(h0, srcs, dsts, z2)
    h1 = _comb(p, h0, dinv)
    q = _hop()(h1, srcs, dsts, z2)
    out = _final(q, h1, dis, b16)
    return out[:N, :C]


# speedup vs baseline: 59.6538x; 1.4067x over previous
import functools

import jax
import jax.numpy as jnp
from jax import lax
from jax.experimental import pallas as pl
from jax.experimental.pallas import tpu as pltpu
from jax.experimental.pallas import tpu_sc as plsc

N = 10000
D = 128
C = 10
E = 320000

L = 16
NPAD = 10240
NSC = 2
NTILE = 16
NW = NSC * NTILE
SL = NPAD // NTILE
CHUNK = 128
EPT = 10240
NCH = EPT // CHUNK
EPAD = NW * EPT
EDT = E // NW
NB = 8


@functools.cache
def _mesh():
    return plsc.VectorSubcoreMesh(core_axis_name="c", subcore_axis_name="s",
                                  num_cores=NSC, num_subcores=NTILE)


def _deg_body(ei_hbm, z1_hbm, degp_hbm, acc_v, idx_v, tmp_v, red_v, rep_v,
              shacc):
    cid = lax.axis_index("c")
    sid = lax.axis_index("s")
    wid = cid * NTILE + sid
    pltpu.sync_copy(z1_hbm, acc_v)
    pltpu.sync_copy(ei_hbm.at[1, pl.ds(wid * EDT, EDT)], idx_v)

    def scat(j, carry):
        for u in range(5):
            idx = idx_v[pl.ds((j * 5 + u) * L, L)]
            cnt, last = plsc.scan_count(idx)
            plsc.addupdate_scatter(acc_v, [idx], cnt.astype(jnp.float32),
                                   mask=last)
        return carry

    lax.fori_loop(0, EDT // L // 5, scat, 0)

    pltpu.sync_copy(acc_v, shacc.at[sid])
    plsc.subcore_barrier()
    for r in range(NTILE):
        pltpu.sync_copy(shacc.at[r, pl.ds(sid * SL, SL)], tmp_v.at[r])

    def red(c, carry):
        s = jnp.zeros((L,), jnp.float32)
        for r in range(NTILE):
            s = s + tmp_v[r, pl.ds(c * L, L)]
        red_v[pl.ds(c * L, L)] = s
        return carry

    lax.fori_loop(0, SL // L, red, 0)

    def rep(c, carry):
        v16 = red_v[pl.ds(c * L, L)]
        for j in range(L):
            rep_v[c * L + j, :] = jnp.full((L,), v16[j], jnp.float32)
        return carry

    lax.fori_loop(0, SL // L, rep, 0)
    pltpu.sync_copy(rep_v, degp_hbm.at[cid, pl.ds(sid * SL, SL)])


@functools.cache
def _deg():
    return pl.kernel(
        _deg_body,
        out_type=jax.ShapeDtypeStruct((NSC, NPAD, L), jnp.float32),
        mesh=_mesh(),
        compiler_params=pltpu.CompilerParams(needs_layout_passes=False,
                                             use_tc_tiling_on_sc=False),
        scratch_types=[
            pltpu.VMEM((NPAD,), jnp.float32),
            pltpu.VMEM((EDT,), jnp.int32),
            pltpu.VMEM((NTILE, SL), jnp.float32),
            pltpu.VMEM((SL,), jnp.float32),
            pltpu.VMEM((SL, L), jnp.float32),
            pltpu.VMEM_SHARED((NTILE, NPAD), jnp.float32),
        ],
    )


def _hop_body(h_hbm, srcs_hbm, dsts_hbm, z2_hbm, p_hbm,
              idxs_v, idxd_v, rows_v, h_spm, acc_s, gsem, ssem):
    cid = lax.axis_index("c")
    sid = lax.axis_index("s")
    wid = cid * NTILE + sid
    off = sid * SL
    pltpu.sync_copy(h_hbm.at[pl.ds(off, SL)], h_spm.at[pl.ds(off, SL)])
    pltpu.sync_copy(z2_hbm, acc_s.at[pl.ds(off, SL)])
    pltpu.sync_copy(srcs_hbm.at[wid], idxs_v)
    pltpu.sync_copy(dsts_hbm.at[wid], idxd_v)
    plsc.subcore_barrier()

    def g_copy(c, b):
        return pltpu.make_async_copy(h_spm.at[idxs_v.at[c]], rows_v.at[b],
                                     gsem.at[b])

    def s_wait(c, b):
        pltpu.make_async_copy(rows_v.at[b], acc_s.at[idxd_v.at[c]],
                              ssem.at[b]).wait()

    def tick(t, lo, hi):
        if lo:
            s_wait(t - 4, (t + 4) % NB)
        if hi:
            g_copy(t + 4, (t + 4) % NB).start()
        g_copy(t, t % NB).wait()
        pltpu.async_copy(rows_v.at[t % NB], acc_s.at[idxd_v.at[t]],
                         ssem.at[t % NB], add=True)

    for t in range(4):
        g_copy(t, t).start()
    for t in range(NB):
        tick(t, t >= 4, True)

    def step(k, carry):
        for b in range(NB):
            t = k * NB + b
            s_wait(t - 4, (b + 4) % NB)
            g_copy(t + 4, (b + 4) % NB).start()
            g_copy(t, b).wait()
            pltpu.async_copy(rows_v.at[b], acc_s.at[idxd_v.at[t]],
                             ssem.at[b], add=True)
        return carry

    lax.fori_loop(1, NCH // NB - 1, step, 0)
    for t in range(NCH - NB, NCH):
        tick(t, True, t + 4 < NCH)
    for c in range(NCH - 4, NCH):
        s_wait(c, c % NB)

    plsc.subcore_barrier()
    pltpu.sync_copy(acc_s.at[pl.ds(off, SL)], p_hbm.at[cid, pl.ds(off, SL)])


@functools.cache
def _hop():
    return pl.kernel(
        _hop_body,
        out_type=jax.ShapeDtypeStruct((NSC, NPAD, L), jnp.float32),
        mesh=_mesh(),
        compiler_params=pltpu.CompilerParams(needs_layout_passes=False,
                                             use_tc_tiling_on_sc=False),
        scratch_types=[
            pltpu.VMEM((NCH, CHUNK), jnp.int32),
            pltpu.VMEM((NCH, CHUNK), jnp.int32),
            pltpu.VMEM((NB, CHUNK, L), jnp.float32),
            pltpu.VMEM_SHARED((NPAD, L), jnp.float32),
            pltpu.VMEM_SHARED((NPAD, L), jnp.float32),
            pltpu.SemaphoreType.DMA((NB,)),
            pltpu.SemaphoreType.DMA((NB,)),
        ],
    )


def _mm_body(x_ref, w_ref, o_ref):
    o_ref[pl.ds(0, N)] = jnp.dot(x_ref[...], w_ref[...],
                                 preferred_element_type=jnp.float32)
    o_ref[pl.ds(N, NPAD - N)] = jnp.zeros((NPAD - N, L), jnp.float32)


def _prep_body(degp_ref, g_ref, h0_ref, dis_ref, dinv_ref):
    deg = degp_ref[0] + degp_ref[1] + 1.0
    dis = lax.rsqrt(deg)
    dinv = 1.0 / deg
    h0_ref[...] = g_ref[...] * dis
    dis_ref[...] = dis
    dinv_ref[...] = dinv


def _comb_body(p_ref, h_ref, s_ref, o_ref):
    o_ref[...] = (p_ref[0] + p_ref[1] + h_ref[...]) * s_ref[...]


def _final_body(q_ref, h_ref, s_ref, b_ref, o_ref):
    o_ref[...] = (q_ref[0] + q_ref[1] + h_ref[...]) * s_ref[...] + b_ref[...]


def _mm(x, wp):
    return pl.pallas_call(
        _mm_body,
        out_shape=jax.ShapeDtypeStruct((NPAD, L), jnp.float32),
    )(x, wp)


def _prep(degp, g):
    return pl.pallas_call(
        _prep_body,
        out_shape=[jax.ShapeDtypeStruct((NPAD, L), jnp.float32)] * 3,
    )(degp, g)


def _comb(p, h, s):
    return pl.pallas_call(
        _comb_body,
        out_shape=jax.ShapeDtypeStruct((NPAD, L), jnp.float32),
    )(p, h, s)


def _final(q, h, s, b16):
    return pl.pallas_call(
        _final_body,
        out_shape=jax.ShapeDtypeStruct((NPAD, L), jnp.float32),
    )(q, h, s, b16)


def kernel(x, edge_index, W, b):
    src = edge_index[0]
    dst = edge_index[1]
    padi = jnp.full((EPAD - E,), N, jnp.int32)
    srcs = jnp.concatenate([src, padi]).reshape(NW, NCH, CHUNK)
    dsts = jnp.concatenate([dst, padi]).reshape(NW, NCH, CHUNK)

    wp = jnp.pad(W, ((0, 0), (0, L - C)))
    b16 = jnp.pad(b, (0, L - C)).reshape(1, L)
    z1 = jnp.zeros((NPAD,), jnp.float32)
    z2 = jnp.zeros((SL, L), jnp.float32)

    degp = _deg()(edge_index, z1)
    g = _mm(x, wp)
    h0, dis, dinv = _prep(degp, g)
    p = _hop()(h0, srcs, dsts, z2)
    h1 = _comb(p, h0, dinv)
    q = _hop()(h1, srcs, dsts, z2)
    out = _final(q, h1, dis, b16)
    return out[:N, :C]


# speedup vs baseline: 74.3687x; 1.2467x over previous
import functools

import jax
import jax.numpy as jnp
from jax import lax
from jax.experimental import pallas as pl
from jax.experimental.pallas import tpu as pltpu
from jax.experimental.pallas import tpu_sc as plsc

N = 10000
D = 128
C = 10
E = 320000

L = 16
NPAD = 10240
NSC = 2
NTILE = 16
NW = NSC * NTILE
SL = NPAD // NTILE
CHUNK = 128
EPT = 10240
NCH = EPT // CHUNK
EPAD = NW * EPT
EDT = E // NW
NB = 8

_SC_PARAMS = None


@functools.cache
def _mesh():
    return plsc.VectorSubcoreMesh(core_axis_name="c", subcore_axis_name="s",
                                  num_cores=NSC, num_subcores=NTILE)


def _params():
    return pltpu.CompilerParams(needs_layout_passes=False,
                                use_tc_tiling_on_sc=False)


def _rsqrt16(x):
    i = plsc.bitcast(x, jnp.int32)
    y = plsc.bitcast(jnp.int32(0x5F3759DF) - (i >> 1), jnp.float32)
    y = y * (1.5 - 0.5 * x * y * y)
    y = y * (1.5 - 0.5 * x * y * y)
    return y * (1.5 - 0.5 * x * y * y)


def _deg_body(ei_hbm, z1_hbm, degp_hbm, acc_v, idx_v, tmp_v, red_v, shacc):
    cid = lax.axis_index("c")
    sid = lax.axis_index("s")
    wid = cid * NTILE + sid
    pltpu.sync_copy(z1_hbm, acc_v)
    pltpu.sync_copy(ei_hbm.at[1, pl.ds(wid * EDT, EDT)], idx_v)

    def scat(j, carry):
        for u in range(5):
            idx = idx_v[pl.ds((j * 5 + u) * L, L)]
            cnt, last = plsc.scan_count(idx)
            plsc.addupdate_scatter(acc_v, [idx], cnt.astype(jnp.float32),
                                   mask=last)
        return carry

    lax.fori_loop(0, EDT // L // 5, scat, 0)

    pltpu.sync_copy(acc_v, shacc.at[sid])
    plsc.subcore_barrier()
    for r in range(NTILE):
        pltpu.sync_copy(shacc.at[r, pl.ds(sid * SL, SL)], tmp_v.at[r])

    def red(c, carry):
        s = jnp.zeros((L,), jnp.float32)
        for r in range(NTILE):
            s = s + tmp_v[r, pl.ds(c * L, L)]
        red_v[pl.ds(c * L, L)] = s
        return carry

    lax.fori_loop(0, SL // L, red, 0)
    pltpu.sync_copy(red_v, degp_hbm.at[cid, pl.ds(sid * SL, SL)])


@functools.cache
def _deg():
    return pl.kernel(
        _deg_body,
        out_type=jax.ShapeDtypeStruct((NSC, NPAD), jnp.float32),
        mesh=_mesh(),
        compiler_params=_params(),
        scratch_types=[
            pltpu.VMEM((NPAD,), jnp.float32),
            pltpu.VMEM((EDT,), jnp.int32),
            pltpu.VMEM((NTILE, SL), jnp.float32),
            pltpu.VMEM((SL,), jnp.float32),
            pltpu.VMEM_SHARED((NTILE, NPAD), jnp.float32),
        ],
    )


def _scale_rows(d0_v, d1_v, h_v, kind, terms):
    def vv(c, carry):
        dv = d0_v[pl.ds(c * L, L)] + d1_v[pl.ds(c * L, L)] + 1.0
        s = _rsqrt16(dv) if kind == "rsqrt" else 1.0 / dv
        for j in range(L):
            n = c * L + j
            row = terms[0][n, :]
            for t in terms[1:]:
                row = row + t[n, :]
            h_v[n, :] = row * jnp.full((L,), s[j], jnp.float32)
        return carry

    lax.fori_loop(0, SL // L, vv, 0)


def _edge_pipeline(idxs_v, idxd_v, rows_v, h_spm, acc_s, gsem, ssem):
    def g_copy(c, b):
        return pltpu.make_async_copy(h_spm.at[idxs_v.at[c]], rows_v.at[b],
                                     gsem.at[b])

    def s_wait(c, b):
        pltpu.make_async_copy(rows_v.at[b], acc_s.at[idxd_v.at[c]],
                              ssem.at[b]).wait()

    def tick(t, lo, hi):
        if lo:
            s_wait(t - 4, (t + 4) % NB)
        if hi:
            g_copy(t + 4, (t + 4) % NB).start()
        g_copy(t, t % NB).wait()
        pltpu.async_copy(rows_v.at[t % NB], acc_s.at[idxd_v.at[t]],
                         ssem.at[t % NB], add=True)

    for t in range(4):
        g_copy(t, t).start()
    for t in range(NB):
        tick(t, t >= 4, True)

    def step(k, carry):
        for b in range(NB):
            t = k * NB + b
            s_wait(t - 4, (b + 4) % NB)
            g_copy(t + 4, (b + 4) % NB).start()
            g_copy(t, b).wait()
            pltpu.async_copy(rows_v.at[b], acc_s.at[idxd_v.at[t]],
                             ssem.at[b], add=True)
        return carry

    lax.fori_loop(1, NCH // NB - 1, step, 0)
    for t in range(NCH - NB, NCH):
        tick(t, True, t + 4 < NCH)
    for c in range(NCH - 4, NCH):
        s_wait(c, c % NB)


def _hop1_body(g_hbm, degp_hbm, srcs_hbm, dsts_hbm, z2_hbm, p_hbm, h0_hbm,
               idxs_v, idxd_v, rows_v, d0_v, d1_v, g_v, h_v,
               h_spm, acc_s, gsem, ssem):
    cid = lax.axis_index("c")
    sid = lax.axis_index("s")
    wid = cid * NTILE + sid
    off = sid * SL
    pltpu.sync_copy(z2_hbm, acc_s.at[pl.ds(off, SL)])
    pltpu.sync_copy(srcs_hbm.at[wid], idxs_v)
    pltpu.sync_copy(dsts_hbm.at[wid], idxd_v)
    pltpu.sync_copy(degp_hbm.at[0, pl.ds(off, SL)], d0_v)
    pltpu.sync_copy(degp_hbm.at[1, pl.ds(off, SL)], d1_v)
    pltpu.sync_copy(g_hbm.at[pl.ds(off, SL)], g_v)
    _scale_rows(d0_v, d1_v, h_v, "rsqrt", [g_v])
    pltpu.sync_copy(h_v, h_spm.at[pl.ds(off, SL)])
    pltpu.sync_copy(h_v, h0_hbm.at[pl.ds(off, SL)])
    plsc.subcore_barrier()
    _edge_pipeline(idxs_v, idxd_v, rows_v, h_spm, acc_s, gsem, ssem)
    plsc.subcore_barrier()
    pltpu.sync_copy(acc_s.at[pl.ds(off, SL)], p_hbm.at[cid, pl.ds(off, SL)])


def _hop2_body(p_hbm, h0_hbm, degp_hbm, srcs_hbm, dsts_hbm, z2_hbm,
               q_hbm, h1_hbm,
               idxs_v, idxd_v, rows_v, d0_v, d1_v, p0_v, p1_v, g_v, h_v,
               h_spm, acc_s, gsem, ssem):
    cid = lax.axis_index("c")
    sid = lax.axis_index("s")
    wid = cid * NTILE + sid
    off = sid * SL
    pltpu.sync_copy(z2_hbm, acc_s.at[pl.ds(off, SL)])
    pltpu.sync_copy(srcs_hbm.at[wid], idxs_v)
    pltpu.sync_copy(dsts_hbm.at[wid], idxd_v)
    pltpu.sync_copy(degp_hbm.at[0, pl.ds(off, SL)], d0_v)
    pltpu.sync_copy(degp_hbm.at[1, pl.ds(off, SL)], d1_v)
    pltpu.sync_copy(p_hbm.at[0, pl.ds(off, SL)], p0_v)
    pltpu.sync_copy(p_hbm.at[1, pl.ds(off, SL)], p1_v)
    pltpu.sync_copy(h0_hbm.at[pl.ds(off, SL)], g_v)
    _scale_rows(d0_v, d1_v, h_v, "inv", [p0_v, p1_v, g_v])
    pltpu.sync_copy(h_v, h_spm.at[pl.ds(off, SL)])
    pltpu.sync_copy(h_v, h1_hbm.at[pl.ds(off, SL)])
    plsc.subcore_barrier()
    _edge_pipeline(idxs_v, idxd_v, rows_v, h_spm, acc_s, gsem, ssem)
    plsc.subcore_barrier()
    pltpu.sync_copy(acc_s.at[pl.ds(off, SL)], q_hbm.at[cid, pl.ds(off, SL)])


def _fin_body(q_hbm, h1_hbm, degp_hbm, b_hbm, out_hbm,
              d0_v, d1_v, p0_v, p1_v, g_v, h_v, b_v):
    cid = lax.axis_index("c")
    sid = lax.axis_index("s")
    off = sid * SL
    hoff = off + cid * (SL // 2)
    HS = SL // 2
    pltpu.sync_copy(degp_hbm.at[0, pl.ds(hoff, HS)], d0_v)
    pltpu.sync_copy(degp_hbm.at[1, pl.ds(hoff, HS)], d1_v)
    pltpu.sync_copy(q_hbm.at[0, pl.ds(hoff, HS)], p0_v)
    pltpu.sync_copy(q_hbm.at[1, pl.ds(hoff, HS)], p1_v)
    pltpu.sync_copy(h1_hbm.at[pl.ds(hoff, HS)], g_v)
    pltpu.sync_copy(b_hbm, b_v)
    bias = b_v[...]

    def vv(c, carry):
        dv = d0_v[pl.ds(c * L, L)] + d1_v[pl.ds(c * L, L)] + 1.0
        s = _rsqrt16(dv)
        for j in range(L):
            n = c * L + j
            row = p0_v[n, :] + p1_v[n, :] + g_v[n, :]
            h_v[n, :] = row * jnp.full((L,), s[j], jnp.float32) + bias
        return carry

    lax.fori_loop(0, HS // L, vv, 0)
    pltpu.sync_copy(h_v, out_hbm.at[pl.ds(hoff, HS)])


@functools.cache
def _hop1():
    return pl.kernel(
        _hop1_body,
        out_type=[jax.ShapeDtypeStruct((NSC, NPAD, L), jnp.float32),
                  jax.ShapeDtypeStruct((NPAD, L), jnp.float32)],
        mesh=_mesh(),
        compiler_params=_params(),
        scratch_types=[
            pltpu.VMEM((NCH, CHUNK), jnp.int32),
            pltpu.VMEM((NCH, CHUNK), jnp.int32),
            pltpu.VMEM((NB, CHUNK, L), jnp.float32),
            pltpu.VMEM((SL,), jnp.float32),
            pltpu.VMEM((SL,), jnp.float32),
            pltpu.VMEM((SL, L), jnp.float32),
            pltpu.VMEM((SL, L), jnp.float32),
            pltpu.VMEM_SHARED((NPAD, L), jnp.float32),
            pltpu.VMEM_SHARED((NPAD, L), jnp.float32),
            pltpu.SemaphoreType.DMA((NB,)),
            pltpu.SemaphoreType.DMA((NB,)),
        ],
    )


@functools.cache
def _hop2():
    return pl.kernel(
        _hop2_body,
        out_type=[jax.ShapeDtypeStruct((NSC, NPAD, L), jnp.float32),
                  jax.ShapeDtypeStruct((NPAD, L), jnp.float32)],
        mesh=_mesh(),
        compiler_params=_params(),
        scratch_types=[
            pltpu.VMEM((NCH, CHUNK), jnp.int32),
            pltpu.VMEM((NCH, CHUNK), jnp.int32),
            pltpu.VMEM((NB, CHUNK, L), jnp.float32),
            pltpu.VMEM((SL,), jnp.float32),
            pltpu.VMEM((SL,), jnp.float32),
            pltpu.VMEM((SL, L), jnp.float32),
            pltpu.VMEM((SL, L), jnp.float32),
            pltpu.VMEM((SL, L), jnp.float32),
            pltpu.VMEM((SL, L), jnp.float32),
            pltpu.VMEM_SHARED((NPAD, L), jnp.float32),
            pltpu.VMEM_SHARED((NPAD, L), jnp.float32),
            pltpu.SemaphoreType.DMA((NB,)),
            pltpu.SemaphoreType.DMA((NB,)),
        ],
    )


@functools.cache
def _fin():
    HS = SL // 2
    return pl.kernel(
        _fin_body,
        out_type=jax.ShapeDtypeStruct((NPAD, L), jnp.float32),
        mesh=_mesh(),
        compiler_params=_params(),
        scratch_types=[
            pltpu.VMEM((HS,), jnp.float32),
            pltpu.VMEM((HS,), jnp.float32),
            pltpu.VMEM((HS, L), jnp.float32),
            pltpu.VMEM((HS, L), jnp.float32),
            pltpu.VMEM((HS, L), jnp.float32),
            pltpu.VMEM((HS, L), jnp.float32),
            pltpu.VMEM((L,), jnp.float32),
        ],
    )


def _mm_body(x_ref, w_ref, o_ref):
    o_ref[pl.ds(0, N)] = jnp.dot(x_ref[...], w_ref[...],
                                 preferred_element_type=jnp.float32)
    o_ref[pl.ds(N, NPAD - N)] = jnp.zeros((NPAD - N, L), jnp.float32)


def _mm(x, wp):
    return pl.pallas_call(
        _mm_body,
        out_shape=jax.ShapeDtypeStruct((NPAD, L), jnp.float32),
    )(x, wp)


def kernel(x, edge_index, W, b):
    src = edge_index[0]
    dst = edge_index[1]
    padi = jnp.full((EPAD - E,), N, jnp.int32)
    srcs = jnp.concatenate([src, padi]).reshape(NW, NCH, CHUNK)
    dsts = jnp.concatenate([dst, padi]).reshape(NW, NCH, CHUNK)

    wp = jnp.pad(W, ((0, 0), (0, L - C)))
    b16 = jnp.pad(b, (0, L - C))
    z1 = jnp.zeros((NPAD,), jnp.float32)
    z2 = jnp.zeros((SL, L), jnp.float32)

    degp = _deg()(edge_index, z1)
    g = _mm(x, wp)
    p, h0 = _hop1()(g, degp, srcs, dsts, z2)
    q, h1 = _hop2()(p, h0, degp, srcs, dsts, z2)
    out = _fin()(q, h1, degp, b16)
    return out[:N, :C]


# speedup vs baseline: 76.1281x; 1.0237x over previous
import functools

import jax
import jax.numpy as jnp
from jax import lax
from jax.experimental import pallas as pl
from jax.experimental.pallas import tpu as pltpu
from jax.experimental.pallas import tpu_sc as plsc

N = 10000
D = 128
C = 10
E = 320000

L = 16
NPAD = 10240
NSC = 2
NTILE = 16
NW = NSC * NTILE
SL = NPAD // NTILE
HS = SL // 2
CHUNK = 128
ROWS = E // CHUNK
RPT = ROWS // NW
NCH = RPT
XTRA = NW * RPT
EDS = E // NTILE
NB = 8


@functools.cache
def _mesh():
    return plsc.VectorSubcoreMesh(core_axis_name="c", subcore_axis_name="s",
                                  num_cores=NSC, num_subcores=NTILE)


def _params():
    return pltpu.CompilerParams(needs_layout_passes=False,
                                use_tc_tiling_on_sc=False)


def _rsqrt16(x):
    i = plsc.bitcast(x, jnp.int32)
    y = plsc.bitcast(jnp.int32(0x5F3759DF) - (i >> 1), jnp.float32)
    y = y * (1.5 - 0.5 * x * y * y)
    y = y * (1.5 - 0.5 * x * y * y)
    return y * (1.5 - 0.5 * x * y * y)


def _scale_rows(d_v, h_v, kind, terms, nrows):
    def vv(c, carry):
        dv = d_v[pl.ds(c * L, L)]
        s = _rsqrt16(dv) if kind == "rsqrt" else 1.0 / dv
        for j in range(L):
            n = c * L + j
            row = terms[0][n, :]
            for t in terms[1:]:
                row = row + t[n, :]
            h_v[n, :] = row * jnp.full((L,), s[j], jnp.float32)
        return carry

    lax.fori_loop(0, nrows // L, vv, 0)


def _edge_pipeline(wid, idxs_v, idxd_v, rows_v, h_spm, acc_s, gsem, ssem):
    def g_copy(c, b):
        return pltpu.make_async_copy(h_spm.at[idxs_v.at[c]], rows_v.at[b],
                                     gsem.at[b])

    def s_wait(c, b):
        pltpu.make_async_copy(rows_v.at[b], acc_s.at[idxd_v.at[c]],
                              ssem.at[b]).wait()

    def tick(t, lo, hi):
        if lo:
            s_wait(t - 4, (t + 4) % NB)
        if hi:
            g_copy(t + 4, (t + 4) % NB).start()
        g_copy(t, t % NB).wait()
        pltpu.async_copy(rows_v.at[t % NB], acc_s.at[idxd_v.at[t]],
                         ssem.at[t % NB], add=True)

    for t in range(4):
        g_copy(t, t).start()
    for t in range(NB):
        tick(t, t >= 4, True)

    def step(k, carry):
        for b in range(NB):
            t = k * NB + b
            s_wait(t - 4, (b + 4) % NB)
            g_copy(t + 4, (b + 4) % NB).start()
            g_copy(t, b).wait()
            pltpu.async_copy(rows_v.at[b], acc_s.at[idxd_v.at[t]],
                             ssem.at[b], add=True)
        return carry

    lax.fori_loop(1, NCH // NB - 1, step, 0)
    for t in range((NCH // NB - 1) * NB, NCH):
        tick(t, True, t + 4 < NCH)
    for c in range(NCH - 4, NCH):
        s_wait(c, c % NB)

    @pl.when(wid < 4)
    def _():
        pltpu.make_async_copy(h_spm.at[idxs_v.at[NCH]], rows_v.at[0],
                              gsem.at[0]).start()
        pltpu.make_async_copy(h_spm.at[idxs_v.at[NCH]], rows_v.at[0],
                              gsem.at[0]).wait()
        pltpu.sync_copy(rows_v.at[0], acc_s.at[idxd_v.at[NCH]], add=True)


def _load_edges(wid, src2d_hbm, dst2d_hbm, idxs_v, idxd_v):
    pltpu.sync_copy(src2d_hbm.at[pl.ds(wid * RPT, RPT)],
                    idxs_v.at[pl.ds(0, RPT)])
    pltpu.sync_copy(dst2d_hbm.at[pl.ds(wid * RPT, RPT)],
                    idxd_v.at[pl.ds(0, RPT)])

    @pl.when(wid < 4)
    def _():
        pltpu.sync_copy(src2d_hbm.at[pl.ds(XTRA + wid, 1)],
                        idxs_v.at[pl.ds(RPT, 1)])
        pltpu.sync_copy(dst2d_hbm.at[pl.ds(XTRA + wid, 1)],
                        idxd_v.at[pl.ds(RPT, 1)])


def _hop1_body(g_hbm, ei_hbm, src2d_hbm, dst2d_hbm, z2_hbm,
               p_hbm, h0_hbm, deg_hbm,
               dacc_v, didx_v, tmp_v, red_v, idxs_v, idxd_v, rows_v,
               g_v, h_v, shacc, h_spm, acc_s, gsem, ssem):
    cid = lax.axis_index("c")
    sid = lax.axis_index("s")
    wid = cid * NTILE + sid
    off = sid * SL
    pltpu.sync_copy(z2_hbm, acc_s.at[pl.ds(off, SL)])
    _load_edges(wid, src2d_hbm, dst2d_hbm, idxs_v, idxd_v)
    pltpu.sync_copy(g_hbm.at[pl.ds(off, SL)], g_v)

    pltpu.sync_copy(z2_hbm, dacc_v)
    pltpu.sync_copy(ei_hbm.at[1, pl.ds(sid * EDS, EDS)], didx_v)
    ones = jnp.ones((L,), jnp.float32)

    def scat(j, carry):
        for u in range(5):
            idx = didx_v[pl.ds((j * 5 + u) * L, L)]
            plsc.addupdate_scatter(dacc_v, [idx >> 4, idx & 15], ones)
        return carry

    lax.fori_loop(0, EDS // L // 5, scat, 0)
    pltpu.sync_copy(dacc_v, shacc.at[sid])
    plsc.subcore_barrier()
    for r in range(NTILE):
        pltpu.sync_copy(shacc.at[r, pl.ds(sid * (SL // L), SL // L)],
                        tmp_v.at[r])

    def red(c, carry):
        s = jnp.full((L,), 1.0, jnp.float32)
        for r in range(NTILE):
            s = s + tmp_v[r, c, :]
        red_v[pl.ds(c * L, L)] = s
        return carry

    lax.fori_loop(0, SL // L, red, 0)
    pltpu.sync_copy(red_v.at[pl.ds(cid * HS, HS)],
                    deg_hbm.at[pl.ds(off + cid * HS, HS)])

    _scale_rows(red_v, h_v, "rsqrt", [g_v], SL)
    pltpu.sync_copy(h_v, h_spm.at[pl.ds(off, SL)])
    pltpu.sync_copy(h_v, h0_hbm.at[pl.ds(off, SL)])
    plsc.subcore_barrier()

    _edge_pipeline(wid, idxs_v, idxd_v, rows_v, h_spm, acc_s, gsem, ssem)
    plsc.subcore_barrier()
    pltpu.sync_copy(acc_s.at[pl.ds(off, SL)], p_hbm.at[cid, pl.ds(off, SL)])


def _hop2_body(p_hbm, h0_hbm, deg_hbm, src2d_hbm, dst2d_hbm, z2_hbm,
               q_hbm, h1_hbm,
               d_v, p0_v, p1_v, g_v, h_v, idxs_v, idxd_v, rows_v,
               h_spm, acc_s, gsem, ssem):
    cid = lax.axis_index("c")
    sid = lax.axis_index("s")
    wid = cid * NTILE + sid
    off = sid * SL
    pltpu.sync_copy(z2_hbm, acc_s.at[pl.ds(off, SL)])
    _load_edges(wid, src2d_hbm, dst2d_hbm, idxs_v, idxd_v)
    pltpu.sync_copy(deg_hbm.at[pl.ds(off, SL)], d_v)
    pltpu.sync_copy(p_hbm.at[0, pl.ds(off, SL)], p0_v)
    pltpu.sync_copy(p_hbm.at[1, pl.ds(off, SL)], p1_v)
    pltpu.sync_copy(h0_hbm.at[pl.ds(off, SL)], g_v)
    _scale_rows(d_v, h_v, "inv", [p0_v, p1_v, g_v], SL)
    pltpu.sync_copy(h_v, h_spm.at[pl.ds(off, SL)])
    pltpu.sync_copy(h_v, h1_hbm.at[pl.ds(off, SL)])
    plsc.subcore_barrier()
    _edge_pipeline(wid, idxs_v, idxd_v, rows_v, h_spm, acc_s, gsem, ssem)
    plsc.subcore_barrier()
    pltpu.sync_copy(acc_s.at[pl.ds(off, SL)], q_hbm.at[cid, pl.ds(off, SL)])


def _fin_body(q_hbm, h1_hbm, deg_hbm, b_hbm, out_hbm,
              d_v, p0_v, p1_v, g_v, h_v, b_v):
    cid = lax.axis_index("c")
    sid = lax.axis_index("s")
    hoff = sid * SL + cid * HS
    pltpu.sync_copy(deg_hbm.at[pl.ds(hoff, HS)], d_v)
    pltpu.sync_copy(q_hbm.at[0, pl.ds(hoff, HS)], p0_v)
    pltpu.sync_copy(q_hbm.at[1, pl.ds(hoff, HS)], p1_v)
    pltpu.sync_copy(h1_hbm.at[pl.ds(hoff, HS)], g_v)
    pltpu.sync_copy(b_hbm, b_v)
    bias = b_v[...]

    def vv(c, carry):
        s = _rsqrt16(d_v[pl.ds(c * L, L)])
        for j in range(L):
            n = c * L + j
            row = p0_v[n, :] + p1_v[n, :] + g_v[n, :]
            h_v[n, :] = row * jnp.full((L,), s[j], jnp.float32) + bias
        return carry

    lax.fori_loop(0, HS // L, vv, 0)
    pltpu.sync_copy(h_v, out_hbm.at[pl.ds(hoff, HS)])


@functools.cache
def _hop1():
    return pl.kernel(
        _hop1_body,
        out_type=[jax.ShapeDtypeStruct((NSC, NPAD, L), jnp.float32),
                  jax.ShapeDtypeStruct((NPAD, L), jnp.float32),
                  jax.ShapeDtypeStruct((NPAD,), jnp.float32)],
        mesh=_mesh(),
        compiler_params=_params(),
        scratch_types=[
            pltpu.VMEM((NPAD // L, L), jnp.float32),
            pltpu.VMEM((EDS,), jnp.int32),
            pltpu.VMEM((NTILE, SL // L, L), jnp.float32),
            pltpu.VMEM((SL,), jnp.float32),
            pltpu.VMEM((RPT + 1, CHUNK), jnp.int32),
            pltpu.VMEM((RPT + 1, CHUNK), jnp.int32),
            pltpu.VMEM((NB, CHUNK, L), jnp.float32),
            pltpu.VMEM((SL, L), jnp.float32),
            pltpu.VMEM((SL, L), jnp.float32),
            pltpu.VMEM_SHARED((NTILE, NPAD // L, L), jnp.float32),
            pltpu.VMEM_SHARED((NPAD, L), jnp.float32),
            pltpu.VMEM_SHARED((NPAD, L), jnp.float32),
            pltpu.SemaphoreType.DMA((NB,)),
            pltpu.SemaphoreType.DMA((NB,)),
        ],
    )


@functools.cache
def _hop2():
    return pl.kernel(
        _hop2_body,
        out_type=[jax.ShapeDtypeStruct((NSC, NPAD, L), jnp.float32),
                  jax.ShapeDtypeStruct((NPAD, L), jnp.float32)],
        mesh=_mesh(),
        compiler_params=_params(),
        scratch_types=[
            pltpu.VMEM((SL,), jnp.float32),
            pltpu.VMEM((SL, L), jnp.float32),
            pltpu.VMEM((SL, L), jnp.float32),
            pltpu.VMEM((SL, L), jnp.float32),
            pltpu.VMEM((SL, L), jnp.float32),
            pltpu.VMEM((RPT + 1, CHUNK), jnp.int32),
            pltpu.VMEM((RPT + 1, CHUNK), jnp.int32),
            pltpu.VMEM((NB, CHUNK, L), jnp.float32),
            pltpu.VMEM_SHARED((NPAD, L), jnp.float32),
            pltpu.VMEM_SHARED((NPAD, L), jnp.float32),
            pltpu.SemaphoreType.DMA((NB,)),
            pltpu.SemaphoreType.DMA((NB,)),
        ],
    )


@functools.cache
def _fin():
    return pl.kernel(
        _fin_body,
        out_type=jax.ShapeDtypeStruct((NPAD, L), jnp.float32),
        mesh=_mesh(),
        compiler_params=_params(),
        scratch_types=[
            pltpu.VMEM((HS,), jnp.float32),
            pltpu.VMEM((HS, L), jnp.float32),
            pltpu.VMEM((HS, L), jnp.float32),
            pltpu.VMEM((HS, L), jnp.float32),
            pltpu.VMEM((HS, L), jnp.float32),
            pltpu.VMEM((L,), jnp.float32),
        ],
    )


def _mm_body(x_ref, w_ref, o_ref):
    o_ref[pl.ds(0, N)] = jnp.dot(x_ref[...], w_ref[...],
                                 preferred_element_type=jnp.float32)
    o_ref[pl.ds(N, NPAD - N)] = jnp.zeros((NPAD - N, L), jnp.float32)


def _mm(x, wp):
    return pl.pallas_call(
        _mm_body,
        out_shape=jax.ShapeDtypeStruct((NPAD, L), jnp.float32),
    )(x, wp)


def kernel(x, edge_index, W, b):
    src2d = edge_index[0].reshape(ROWS, CHUNK)
    dst2d = edge_index[1].reshape(ROWS, CHUNK)

    wp = jnp.pad(W, ((0, 0), (0, L - C)))
    b16 = jnp.pad(b, (0, L - C))
    z2 = jnp.zeros((SL, L), jnp.float32)

    g = _mm(x, wp)
    p, h0, deg = _hop1()(g, edge_index, src2d, dst2d, z2)
    q, h1 = _hop2()(p, h0, deg, src2d, dst2d, z2)
    out = _fin()(q, h1, deg, b16)
    return out[:N, :C]


# speedup vs baseline: 90.4337x; 1.1879x over previous
import functools

import jax
import jax.numpy as jnp
from jax import lax
from jax.experimental import pallas as pl
from jax.experimental.pallas import tpu as pltpu
from jax.experimental.pallas import tpu_sc as plsc

N = 10000
D = 128
C = 10
E = 320000

L = 16
NPAD = 10240
NSC = 2
NTILE = 16
NW = NSC * NTILE
SL = NPAD // NTILE
HS = SL // 2
CHUNK = 128
ROWS = E // CHUNK
RPT = ROWS // NW
NCH = RPT
XTRA = NW * RPT
EDS = E // NTILE
NB = 8


@functools.cache
def _mesh():
    return plsc.VectorSubcoreMesh(core_axis_name="c", subcore_axis_name="s",
                                  num_cores=NSC, num_subcores=NTILE)


def _params():
    return pltpu.CompilerParams(needs_layout_passes=False,
                                use_tc_tiling_on_sc=False)


def _rsqrt16(x):
    i = plsc.bitcast(x, jnp.int32)
    y = plsc.bitcast(jnp.int32(0x5F3759DF) - (i >> 1), jnp.float32)
    y = y * (1.5 - 0.5 * x * y * y)
    y = y * (1.5 - 0.5 * x * y * y)
    return y * (1.5 - 0.5 * x * y * y)


def _scale_rows(d_v, h_v, kind, terms, nrows):
    def vv(c, carry):
        dv = d_v[pl.ds(c * L, L)]
        s = _rsqrt16(dv) if kind == "rsqrt" else 1.0 / dv
        for j in range(L):
            n = c * L + j
            row = terms[0][n, :]
            for t in terms[1:]:
                row = row + t[n, :]
            h_v[n, :] = row * jnp.full((L,), s[j], jnp.float32)
        return carry

    lax.fori_loop(0, nrows // L, vv, 0)


def _edge_pipeline(wid, idxs_v, idxd_v, rows_v, h_spm, acc_s, gsem, ssem):
    def g_copy(c, b):
        return pltpu.make_async_copy(h_spm.at[idxs_v.at[c]], rows_v.at[b],
                                     gsem.at[b])

    def s_wait(c, b):
        pltpu.make_async_copy(rows_v.at[b], acc_s.at[idxd_v.at[c]],
                              ssem.at[b]).wait()

    def tick(t, lo, hi):
        if lo:
            s_wait(t - 4, (t + 4) % NB)
        if hi:
            g_copy(t + 4, (t + 4) % NB).start()
        g_copy(t, t % NB).wait()
        pltpu.async_copy(rows_v.at[t % NB], acc_s.at[idxd_v.at[t]],
                         ssem.at[t % NB], add=True)

    for t in range(4):
        g_copy(t, t).start()
    for t in range(NB):
        tick(t, t >= 4, True)

    def step(k, carry):
        for b in range(NB):
            t = k * NB + b
            s_wait(t - 4, (b + 4) % NB)
            g_copy(t + 4, (b + 4) % NB).start()
            g_copy(t, b).wait()
            pltpu.async_copy(rows_v.at[b], acc_s.at[idxd_v.at[t]],
                             ssem.at[b], add=True)
        return carry

    lax.fori_loop(1, NCH // NB - 1, step, 0)
    for t in range((NCH // NB - 1) * NB, NCH):
        tick(t, True, t + 4 < NCH)
    for c in range(NCH - 4, NCH):
        s_wait(c, c % NB)

    @pl.when(wid < 4)
    def _():
        pltpu.make_async_copy(h_spm.at[idxs_v.at[NCH]], rows_v.at[0],
                              gsem.at[0]).start()
        pltpu.make_async_copy(h_spm.at[idxs_v.at[NCH]], rows_v.at[0],
                              gsem.at[0]).wait()
        pltpu.sync_copy(rows_v.at[0], acc_s.at[idxd_v.at[NCH]], add=True)


def _load_edges(wid, ei3_hbm, idxs_v, idxd_v):
    pltpu.sync_copy(ei3_hbm.at[0, pl.ds(wid * RPT, RPT)],
                    idxs_v.at[pl.ds(0, RPT)])
    pltpu.sync_copy(ei3_hbm.at[1, pl.ds(wid * RPT, RPT)],
                    idxd_v.at[pl.ds(0, RPT)])

    @pl.when(wid < 4)
    def _():
        pltpu.sync_copy(ei3_hbm.at[0, pl.ds(XTRA + wid, 1)],
                        idxs_v.at[pl.ds(RPT, 1)])
        pltpu.sync_copy(ei3_hbm.at[1, pl.ds(XTRA + wid, 1)],
                        idxd_v.at[pl.ds(RPT, 1)])


def _deg_body(ei3_hbm, z2_hbm, degp_hbm, dacc_v, didx_v, tmp_v, red_v, shacc):
    cid = lax.axis_index("c")
    sid = lax.axis_index("s")
    wid = cid * NTILE + sid
    pltpu.sync_copy(z2_hbm, dacc_v)
    pltpu.sync_copy(ei3_hbm.at[1, pl.ds(wid * RPT, RPT)],
                    didx_v.at[pl.ds(0, RPT)])

    @pl.when(wid < 4)
    def _():
        pltpu.sync_copy(ei3_hbm.at[1, pl.ds(XTRA + wid, 1)],
                        didx_v.at[pl.ds(RPT, 1)])

    ones = jnp.ones((L,), jnp.float32)

    def one_row(r):
        for u in range(CHUNK // L):
            idx = didx_v[r, pl.ds(u * L, L)]
            plsc.addupdate_scatter(dacc_v, [idx >> 4, idx & 15], ones)

    def scat(r, carry):
        one_row(r)
        return carry

    lax.fori_loop(0, RPT, scat, 0)

    @pl.when(wid < 4)
    def _():
        one_row(RPT)

    pltpu.sync_copy(dacc_v, shacc.at[sid])
    plsc.subcore_barrier()
    for r in range(NTILE):
        pltpu.sync_copy(shacc.at[r, pl.ds(sid * (SL // L), SL // L)],
                        tmp_v.at[r])

    def red(c, carry):
        s = jnp.zeros((L,), jnp.float32)
        for r in range(NTILE):
            s = s + tmp_v[r, c, :]
        red_v[pl.ds(c * L, L)] = s
        return carry

    lax.fori_loop(0, SL // L, red, 0)
    pltpu.sync_copy(red_v, degp_hbm.at[cid, pl.ds(sid * SL, SL)])


@functools.cache
def _deg():
    return pl.kernel(
        _deg_body,
        out_type=jax.ShapeDtypeStruct((NSC, NPAD), jnp.float32),
        mesh=_mesh(),
        compiler_params=_params(),
        scratch_types=[
            pltpu.VMEM((NPAD // L, L), jnp.float32),
            pltpu.VMEM((RPT + 1, CHUNK), jnp.int32),
            pltpu.VMEM((NTILE, SL // L, L), jnp.float32),
            pltpu.VMEM((SL,), jnp.float32),
            pltpu.VMEM_SHARED((NTILE, NPAD // L, L), jnp.float32),
        ],
    )


def _hop1_body(g_hbm, degp_hbm, ei3_hbm, z2_hbm,
               p_hbm, h0_hbm, deg_hbm,
               d0_v, d1_v, red_v, idxs_v, idxd_v, rows_v,
               g_v, h_v, h_spm, acc_s, gsem, ssem):
    cid = lax.axis_index("c")
    sid = lax.axis_index("s")
    wid = cid * NTILE + sid
    off = sid * SL
    pltpu.sync_copy(z2_hbm, acc_s.at[pl.ds(off, SL)])
    _load_edges(wid, ei3_hbm, idxs_v, idxd_v)
    pltpu.sync_copy(g_hbm.at[pl.ds(off, SL)], g_v)
    pltpu.sync_copy(degp_hbm.at[0, pl.ds(off, SL)], d0_v)
    pltpu.sync_copy(degp_hbm.at[1, pl.ds(off, SL)], d1_v)

    def dsum(c, carry):
        red_v[pl.ds(c * L, L)] = (d0_v[pl.ds(c * L, L)] +
                                  d1_v[pl.ds(c * L, L)] + 1.0)
        return carry

    lax.fori_loop(0, SL // L, dsum, 0)
    pltpu.sync_copy(red_v.at[pl.ds(cid * HS, HS)],
                    deg_hbm.at[pl.ds(off + cid * HS, HS)])

    _scale_rows(red_v, h_v, "rsqrt", [g_v], SL)
    pltpu.sync_copy(h_v, h_spm.at[pl.ds(off, SL)])
    pltpu.sync_copy(h_v, h0_hbm.at[pl.ds(off, SL)])
    plsc.subcore_barrier()

    _edge_pipeline(wid, idxs_v, idxd_v, rows_v, h_spm, acc_s, gsem, ssem)
    plsc.subcore_barrier()
    pltpu.sync_copy(acc_s.at[pl.ds(off, SL)], p_hbm.at[cid, pl.ds(off, SL)])


def _hop2_body(p_hbm, h0_hbm, deg_hbm, ei3_hbm, z2_hbm,
               q_hbm, h1_hbm,
               d_v, p0_v, p1_v, g_v, h_v, idxs_v, idxd_v, rows_v,
               h_spm, acc_s, gsem, ssem):
    cid = lax.axis_index("c")
    sid = lax.axis_index("s")
    wid = cid * NTILE + sid
    off = sid * SL
    pltpu.sync_copy(z2_hbm, acc_s.at[pl.ds(off, SL)])
    _load_edges(wid, ei3_hbm, idxs_v, idxd_v)
    pltpu.sync_copy(deg_hbm.at[pl.ds(off, SL)], d_v)
    pltpu.sync_copy(p_hbm.at[0, pl.ds(off, SL)], p0_v)
    pltpu.sync_copy(p_hbm.at[1, pl.ds(off, SL)], p1_v)
    pltpu.sync_copy(h0_hbm.at[pl.ds(off, SL)], g_v)
    _scale_rows(d_v, h_v, "inv", [p0_v, p1_v, g_v], SL)
    pltpu.sync_copy(h_v, h_spm.at[pl.ds(off, SL)])
    pltpu.sync_copy(h_v, h1_hbm.at[pl.ds(off, SL)])
    plsc.subcore_barrier()
    _edge_pipeline(wid, idxs_v, idxd_v, rows_v, h_spm, acc_s, gsem, ssem)
    plsc.subcore_barrier()
    pltpu.sync_copy(acc_s.at[pl.ds(off, SL)], q_hbm.at[cid, pl.ds(off, SL)])


def _fin_body(q_hbm, h1_hbm, deg_hbm, b_hbm, out_hbm,
              d_v, p0_v, p1_v, g_v, h_v, b_v):
    cid = lax.axis_index("c")
    sid = lax.axis_index("s")
    hoff = sid * SL + cid * HS
    pltpu.sync_copy(deg_hbm.at[pl.ds(hoff, HS)], d_v)
    pltpu.sync_copy(q_hbm.at[0, pl.ds(hoff, HS)], p0_v)
    pltpu.sync_copy(q_hbm.at[1, pl.ds(hoff, HS)], p1_v)
    pltpu.sync_copy(h1_hbm.at[pl.ds(hoff, HS)], g_v)
    pltpu.sync_copy(b_hbm, b_v)
    bias = b_v[...]

    def vv(c, carry):
        s = _rsqrt16(d_v[pl.ds(c * L, L)])
        for j in range(L):
            n = c * L + j
            row = p0_v[n, :] + p1_v[n, :] + g_v[n, :]
            h_v[n, :] = row * jnp.full((L,), s[j], jnp.float32) + bias
        return carry

    lax.fori_loop(0, HS // L, vv, 0)
    pltpu.sync_copy(h_v, out_hbm.at[pl.ds(hoff, HS)])


@functools.cache
def _hop1():
    return pl.kernel(
        _hop1_body,
        out_type=[jax.ShapeDtypeStruct((NSC, NPAD, L), jnp.float32),
                  jax.ShapeDtypeStruct((NPAD, L), jnp.float32),
                  jax.ShapeDtypeStruct((NPAD,), jnp.float32)],
        mesh=_mesh(),
        compiler_params=_params(),
        scratch_types=[
            pltpu.VMEM((SL,), jnp.float32),
            pltpu.VMEM((SL,), jnp.float32),
            pltpu.VMEM((SL,), jnp.float32),
            pltpu.VMEM((RPT + 1, CHUNK), jnp.int32),
            pltpu.VMEM((RPT + 1, CHUNK), jnp.int32),
            pltpu.VMEM((NB, CHUNK, L), jnp.float32),
            pltpu.VMEM((SL, L), jnp.float32),
            pltpu.VMEM((SL, L), jnp.float32),
            pltpu.VMEM_SHARED((NPAD, L), jnp.float32),
            pltpu.VMEM_SHARED((NPAD, L), jnp.float32),
            pltpu.SemaphoreType.DMA((NB,)),
            pltpu.SemaphoreType.DMA((NB,)),
        ],
    )


@functools.cache
def _hop2():
    return pl.kernel(
        _hop2_body,
        out_type=[jax.ShapeDtypeStruct((NSC, NPAD, L), jnp.float32),
                  jax.ShapeDtypeStruct((NPAD, L), jnp.float32)],
        mesh=_mesh(),
        compiler_params=_params(),
        scratch_types=[
            pltpu.VMEM((SL,), jnp.float32),
            pltpu.VMEM((SL, L), jnp.float32),
            pltpu.VMEM((SL, L), jnp.float32),
            pltpu.VMEM((SL, L), jnp.float32),
            pltpu.VMEM((SL, L), jnp.float32),
            pltpu.VMEM((RPT + 1, CHUNK), jnp.int32),
            pltpu.VMEM((RPT + 1, CHUNK), jnp.int32),
            pltpu.VMEM((NB, CHUNK, L), jnp.float32),
            pltpu.VMEM_SHARED((NPAD, L), jnp.float32),
            pltpu.VMEM_SHARED((NPAD, L), jnp.float32),
            pltpu.SemaphoreType.DMA((NB,)),
            pltpu.SemaphoreType.DMA((NB,)),
        ],
    )


@functools.cache
def _fin():
    return pl.kernel(
        _fin_body,
        out_type=jax.ShapeDtypeStruct((NPAD, L), jnp.float32),
        mesh=_mesh(),
        compiler_params=_params(),
        scratch_types=[
            pltpu.VMEM((HS,), jnp.float32),
            pltpu.VMEM((HS, L), jnp.float32),
            pltpu.VMEM((HS, L), jnp.float32),
            pltpu.VMEM((HS, L), jnp.float32),
            pltpu.VMEM((HS, L), jnp.float32),
            pltpu.VMEM((L,), jnp.float32),
        ],
    )


def _mm_body(x_ref, w_ref, o_ref):
    o_ref[pl.ds(0, N)] = jnp.dot(x_ref[...], w_ref[...],
                                 preferred_element_type=jnp.float32)
    o_ref[pl.ds(N, NPAD - N)] = jnp.zeros((NPAD - N, L), jnp.float32)


def _mm(x, wp):
    return pl.pallas_call(
        _mm_body,
        out_shape=jax.ShapeDtypeStruct((NPAD, L), jnp.float32),
    )(x, wp)


def kernel(x, edge_index, W, b):
    ei3 = edge_index.reshape(2, ROWS, CHUNK)

    wp = jnp.pad(W, ((0, 0), (0, L - C)))
    b16 = jnp.pad(b, (0, L - C))
    z2 = jnp.zeros((SL, L), jnp.float32)

    degp = _deg()(ei3, z2)
    g = _mm(x, wp)
    p, h0, deg = _hop1()(g, degp, ei3, z2)
    q, h1 = _hop2()(p, h0, deg, ei3, z2)
    out = _fin()(q, h1, deg, b16)
    return out[:N, :C]


# speedup vs baseline: 97.5803x; 1.0790x over previous
import functools

import jax
import jax.numpy as jnp
from jax import lax
from jax.experimental import pallas as pl
from jax.experimental.pallas import tpu as pltpu
from jax.experimental.pallas import tpu_sc as plsc

N = 10000
D = 128
C = 10
E = 320000

L = 16
NPAD = 10240
NSC = 2
NTILE = 16
NW = NSC * NTILE
SL = NPAD // NTILE
HS = SL // 2
CHUNK = 128
ROWS = E // CHUNK
RPT = ROWS // NW
NCH = RPT
XTRA = NW * RPT
EDS = E // NTILE
NB = 8


@functools.cache
def _mesh():
    return plsc.VectorSubcoreMesh(core_axis_name="c", subcore_axis_name="s",
                                  num_cores=NSC, num_subcores=NTILE)


def _params():
    return pltpu.CompilerParams(needs_layout_passes=False,
                                use_tc_tiling_on_sc=False)


def _rsqrt16(x):
    i = plsc.bitcast(x, jnp.int32)
    y = plsc.bitcast(jnp.int32(0x5F3759DF) - (i >> 1), jnp.float32)
    y = y * (1.5 - 0.5 * x * y * y)
    y = y * (1.5 - 0.5 * x * y * y)
    return y * (1.5 - 0.5 * x * y * y)


def _scale_rows(d_v, h_v, kind, terms, nrows):
    def vv(c, carry):
        dv = d_v[pl.ds(c * L, L)]
        s = _rsqrt16(dv) if kind == "rsqrt" else 1.0 / dv
        for j in range(L):
            n = c * L + j
            row = terms[0][n, :]
            for t in terms[1:]:
                row = row + t[n, :]
            h_v[n, :] = row * jnp.full((L,), s[j], jnp.float32)
        return carry

    lax.fori_loop(0, nrows // L, vv, 0)


def _edge_pipeline(wid, idxs_v, idxd_v, rows_v, h_spm, acc_s, gsem, ssem):
    def g_copy(c, b):
        return pltpu.make_async_copy(h_spm.at[idxs_v.at[c]], rows_v.at[b],
                                     gsem.at[b])

    def s_wait(c, b):
        pltpu.make_async_copy(rows_v.at[b], acc_s.at[idxd_v.at[c]],
                              ssem.at[b]).wait()

    def tick(t, lo, hi):
        if lo:
            s_wait(t - 4, (t + 4) % NB)
        if hi:
            g_copy(t + 4, (t + 4) % NB).start()
        g_copy(t, t % NB).wait()
        pltpu.async_copy(rows_v.at[t % NB], acc_s.at[idxd_v.at[t]],
                         ssem.at[t % NB], add=True)

    for t in range(4):
        g_copy(t, t).start()
    for t in range(NB):
        tick(t, t >= 4, True)

    def step(k, carry):
        for b in range(NB):
            t = k * NB + b
            s_wait(t - 4, (b + 4) % NB)
            g_copy(t + 4, (b + 4) % NB).start()
            g_copy(t, b).wait()
            pltpu.async_copy(rows_v.at[b], acc_s.at[idxd_v.at[t]],
                             ssem.at[b], add=True)
        return carry

    lax.fori_loop(1, NCH // NB - 1, step, 0)
    for t in range((NCH // NB - 1) * NB, NCH):
        tick(t, True, t + 4 < NCH)
    for c in range(NCH - 4, NCH):
        s_wait(c, c % NB)

    @pl.when(wid < 4)
    def _():
        pltpu.make_async_copy(h_spm.at[idxs_v.at[NCH]], rows_v.at[0],
                              gsem.at[0]).start()
        pltpu.make_async_copy(h_spm.at[idxs_v.at[NCH]], rows_v.at[0],
                              gsem.at[0]).wait()
        pltpu.sync_copy(rows_v.at[0], acc_s.at[idxd_v.at[NCH]], add=True)


def _load_edges_start(wid, ei3_hbm, idxs_v, idxd_v, sem):
    pltpu.make_async_copy(ei3_hbm.at[0, pl.ds(wid * RPT, RPT)],
                          idxs_v.at[pl.ds(0, RPT)], sem.at[0]).start()
    pltpu.make_async_copy(ei3_hbm.at[1, pl.ds(wid * RPT, RPT)],
                          idxd_v.at[pl.ds(0, RPT)], sem.at[1]).start()

    @pl.when(wid < 4)
    def _():
        pltpu.make_async_copy(ei3_hbm.at[0, pl.ds(XTRA + wid, 1)],
                              idxs_v.at[pl.ds(RPT, 1)], sem.at[2]).start()
        pltpu.make_async_copy(ei3_hbm.at[1, pl.ds(XTRA + wid, 1)],
                              idxd_v.at[pl.ds(RPT, 1)], sem.at[2]).start()


def _load_edges_wait(wid, ei3_hbm, idxs_v, idxd_v, sem):
    pltpu.make_async_copy(ei3_hbm.at[0, pl.ds(wid * RPT, RPT)],
                          idxs_v.at[pl.ds(0, RPT)], sem.at[0]).wait()
    pltpu.make_async_copy(ei3_hbm.at[1, pl.ds(wid * RPT, RPT)],
                          idxd_v.at[pl.ds(0, RPT)], sem.at[1]).wait()

    @pl.when(wid < 4)
    def _():
        pltpu.make_async_copy(ei3_hbm.at[0, pl.ds(XTRA + wid, 1)],
                              idxs_v.at[pl.ds(RPT, 1)], sem.at[2]).wait()
        pltpu.make_async_copy(ei3_hbm.at[1, pl.ds(XTRA + wid, 1)],
                              idxd_v.at[pl.ds(RPT, 1)], sem.at[2]).wait()


def _deg_body(ei3_hbm, z2_hbm, degp_hbm, dacc_v, didx_v, tmp_v, red_v, shacc,
              dsem):
    cid = lax.axis_index("c")
    sid = lax.axis_index("s")
    wid = cid * NTILE + sid
    pltpu.sync_copy(z2_hbm, dacc_v)
    pltpu.sync_copy(ei3_hbm.at[1, pl.ds(wid * RPT, RPT)],
                    didx_v.at[pl.ds(0, RPT)])

    @pl.when(wid < 4)
    def _():
        pltpu.sync_copy(ei3_hbm.at[1, pl.ds(XTRA + wid, 1)],
                        didx_v.at[pl.ds(RPT, 1)])

    ones = jnp.ones((L,), jnp.float32)

    def one_row(r):
        for u in range(CHUNK // L):
            idx = didx_v[r, pl.ds(u * L, L)]
            plsc.addupdate_scatter(dacc_v, [idx >> 4, idx & 15], ones)

    def scat(r, carry):
        one_row(r)
        return carry

    lax.fori_loop(0, RPT, scat, 0)

    @pl.when(wid < 4)
    def _():
        one_row(RPT)

    pltpu.sync_copy(dacc_v, shacc.at[sid])
    plsc.subcore_barrier()

    def t_copy(r):
        return pltpu.make_async_copy(
            shacc.at[r, pl.ds(sid * (SL // L), SL // L)], tmp_v.at[r], dsem)

    for r in range(NTILE):
        t_copy(r).start()
    for r in range(NTILE):
        t_copy(r).wait()

    def red(c, carry):
        s = jnp.zeros((L,), jnp.float32)
        for r in range(NTILE):
            s = s + tmp_v[r, c, :]
        red_v[pl.ds(c * L, L)] = s
        return carry

    lax.fori_loop(0, SL // L, red, 0)
    pltpu.sync_copy(red_v, degp_hbm.at[cid, pl.ds(sid * SL, SL)])


@functools.cache
def _deg():
    return pl.kernel(
        _deg_body,
        out_type=jax.ShapeDtypeStruct((NSC, NPAD), jnp.float32),
        mesh=_mesh(),
        compiler_params=_params(),
        scratch_types=[
            pltpu.VMEM((NPAD // L, L), jnp.float32),
            pltpu.VMEM((RPT + 1, CHUNK), jnp.int32),
            pltpu.VMEM((NTILE, SL // L, L), jnp.float32),
            pltpu.VMEM((SL,), jnp.float32),
            pltpu.VMEM_SHARED((NTILE, NPAD // L, L), jnp.float32),
            pltpu.SemaphoreType.DMA,
        ],
    )


def _hop1_body(g_hbm, degp_hbm, ei3_hbm, z2_hbm,
               p_hbm, h0_hbm, deg_hbm,
               d0_v, d1_v, red_v, idxs_v, idxd_v, rows_v,
               g_v, h_v, h_spm, acc_s, gsem, ssem):
    cid = lax.axis_index("c")
    sid = lax.axis_index("s")
    wid = cid * NTILE + sid
    off = sid * SL
    pltpu.make_async_copy(z2_hbm, acc_s.at[pl.ds(off, SL)], ssem.at[0]).start()
    _load_edges_start(wid, ei3_hbm, idxs_v, idxd_v, gsem)
    pltpu.sync_copy(g_hbm.at[pl.ds(off, SL)], g_v)
    pltpu.sync_copy(degp_hbm.at[0, pl.ds(off, SL)], d0_v)
    pltpu.sync_copy(degp_hbm.at[1, pl.ds(off, SL)], d1_v)

    def dsum(c, carry):
        red_v[pl.ds(c * L, L)] = (d0_v[pl.ds(c * L, L)] +
                                  d1_v[pl.ds(c * L, L)] + 1.0)
        return carry

    lax.fori_loop(0, SL // L, dsum, 0)
    pltpu.sync_copy(red_v.at[pl.ds(cid * HS, HS)],
                    deg_hbm.at[pl.ds(off + cid * HS, HS)])

    _scale_rows(red_v, h_v, "rsqrt", [g_v], SL)
    pltpu.sync_copy(h_v, h_spm.at[pl.ds(off, SL)])
    pltpu.sync_copy(h_v, h0_hbm.at[pl.ds(off, SL)])
    pltpu.make_async_copy(z2_hbm, acc_s.at[pl.ds(off, SL)], ssem.at[0]).wait()
    _load_edges_wait(wid, ei3_hbm, idxs_v, idxd_v, gsem)
    plsc.subcore_barrier()

    _edge_pipeline(wid, idxs_v, idxd_v, rows_v, h_spm, acc_s, gsem, ssem)
    plsc.subcore_barrier()
    pltpu.sync_copy(acc_s.at[pl.ds(off, SL)], p_hbm.at[cid, pl.ds(off, SL)])


def _hop2_body(p_hbm, h0_hbm, deg_hbm, ei3_hbm, z2_hbm,
               q_hbm, h1_hbm,
               d_v, p0_v, p1_v, g_v, h_v, idxs_v, idxd_v, rows_v,
               h_spm, acc_s, gsem, ssem):
    cid = lax.axis_index("c")
    sid = lax.axis_index("s")
    wid = cid * NTILE + sid
    off = sid * SL
    pltpu.make_async_copy(z2_hbm, acc_s.at[pl.ds(off, SL)], ssem.at[0]).start()
    _load_edges_start(wid, ei3_hbm, idxs_v, idxd_v, gsem)
    pltpu.sync_copy(deg_hbm.at[pl.ds(off, SL)], d_v)
    pltpu.sync_copy(p_hbm.at[0, pl.ds(off, SL)], p0_v)
    pltpu.sync_copy(p_hbm.at[1, pl.ds(off, SL)], p1_v)
    pltpu.sync_copy(h0_hbm.at[pl.ds(off, SL)], g_v)
    _scale_rows(d_v, h_v, "inv", [p0_v, p1_v, g_v], SL)
    pltpu.sync_copy(h_v, h_spm.at[pl.ds(off, SL)])
    pltpu.sync_copy(h_v, h1_hbm.at[pl.ds(off, SL)])
    pltpu.make_async_copy(z2_hbm, acc_s.at[pl.ds(off, SL)], ssem.at[0]).wait()
    _load_edges_wait(wid, ei3_hbm, idxs_v, idxd_v, gsem)
    plsc.subcore_barrier()
    _edge_pipeline(wid, idxs_v, idxd_v, rows_v, h_spm, acc_s, gsem, ssem)
    plsc.subcore_barrier()
    pltpu.sync_copy(acc_s.at[pl.ds(off, SL)], q_hbm.at[cid, pl.ds(off, SL)])


def _fin_body(q_hbm, h1_hbm, deg_hbm, b_hbm, out_hbm,
              d_v, p0_v, p1_v, g_v, h_v, b_v):
    cid = lax.axis_index("c")
    sid = lax.axis_index("s")
    hoff = sid * SL + cid * HS
    pltpu.sync_copy(deg_hbm.at[pl.ds(hoff, HS)], d_v)
    pltpu.sync_copy(q_hbm.at[0, pl.ds(hoff, HS)], p0_v)
    pltpu.sync_copy(q_hbm.at[1, pl.ds(hoff, HS)], p1_v)
    pltpu.sync_copy(h1_hbm.at[pl.ds(hoff, HS)], g_v)
    pltpu.sync_copy(b_hbm, b_v)
    bias = b_v[...]

    def vv(c, carry):
        s = _rsqrt16(d_v[pl.ds(c * L, L)])
        for j in range(L):
            n = c * L + j
            row = p0_v[n, :] + p1_v[n, :] + g_v[n, :]
            h_v[n, :] = row * jnp.full((L,), s[j], jnp.float32) + bias
        return carry

    lax.fori_loop(0, HS // L, vv, 0)
    pltpu.sync_copy(h_v, out_hbm.at[pl.ds(hoff, HS)])


@functools.cache
def _hop1():
    return pl.kernel(
        _hop1_body,
        out_type=[jax.ShapeDtypeStruct((NSC, NPAD, L), jnp.float32),
                  jax.ShapeDtypeStruct((NPAD, L), jnp.float32),
                  jax.ShapeDtypeStruct((NPAD,), jnp.float32)],
        mesh=_mesh(),
        compiler_params=_params(),
        scratch_types=[
            pltpu.VMEM((SL,), jnp.float32),
            pltpu.VMEM((SL,), jnp.float32),
            pltpu.VMEM((SL,), jnp.float32),
            pltpu.VMEM((RPT + 1, CHUNK), jnp.int32),
            pltpu.VMEM((RPT + 1, CHUNK), jnp.int32),
            pltpu.VMEM((NB, CHUNK, L), jnp.float32),
            pltpu.VMEM((SL, L), jnp.float32),
            pltpu.VMEM((SL, L), jnp.float32),
            pltpu.VMEM_SHARED((NPAD, L), jnp.float32),
            pltpu.VMEM_SHARED((NPAD, L), jnp.float32),
            pltpu.SemaphoreType.DMA((NB,)),
            pltpu.SemaphoreType.DMA((NB,)),
        ],
    )


@functools.cache
def _hop2():
    return pl.kernel(
        _hop2_body,
        out_type=[jax.ShapeDtypeStruct((NSC, NPAD, L), jnp.float32),
                  jax.ShapeDtypeStruct((NPAD, L), jnp.float32)],
        mesh=_mesh(),
        compiler_params=_params(),
        scratch_types=[
            pltpu.VMEM((SL,), jnp.float32),
            pltpu.VMEM((SL, L), jnp.float32),
            pltpu.VMEM((SL, L), jnp.float32),
            pltpu.VMEM((SL, L), jnp.float32),
            pltpu.VMEM((SL, L), jnp.float32),
            pltpu.VMEM((RPT + 1, CHUNK), jnp.int32),
            pltpu.VMEM((RPT + 1, CHUNK), jnp.int32),
            pltpu.VMEM((NB, CHUNK, L), jnp.float32),
            pltpu.VMEM_SHARED((NPAD, L), jnp.float32),
            pltpu.VMEM_SHARED((NPAD, L), jnp.float32),
            pltpu.SemaphoreType.DMA((NB,)),
            pltpu.SemaphoreType.DMA((NB,)),
        ],
    )


@functools.cache
def _fin():
    return pl.kernel(
        _fin_body,
        out_type=jax.ShapeDtypeStruct((NPAD, L), jnp.float32),
        mesh=_mesh(),
        compiler_params=_params(),
        scratch_types=[
            pltpu.VMEM((HS,), jnp.float32),
            pltpu.VMEM((HS, L), jnp.float32),
            pltpu.VMEM((HS, L), jnp.float32),
            pltpu.VMEM((HS, L), jnp.float32),
            pltpu.VMEM((HS, L), jnp.float32),
            pltpu.VMEM((L,), jnp.float32),
        ],
    )


def _mm_body(x_ref, w_ref, o_ref):
    o_ref[pl.ds(0, N)] = jnp.dot(x_ref[...], w_ref[...],
                                 preferred_element_type=jnp.float32)
    o_ref[pl.ds(N, NPAD - N)] = jnp.zeros((NPAD - N, L), jnp.float32)


def _mm(x, wp):
    return pl.pallas_call(
        _mm_body,
        out_shape=jax.ShapeDtypeStruct((NPAD, L), jnp.float32),
    )(x, wp)


def kernel(x, edge_index, W, b):
    ei3 = edge_index.reshape(2, ROWS, CHUNK)

    wp = jnp.pad(W, ((0, 0), (0, L - C)))
    b16 = jnp.pad(b, (0, L - C))
    z2 = jnp.zeros((SL, L), jnp.float32)

    degp = _deg()(ei3, z2)
    g = _mm(x, wp)
    p, h0, deg = _hop1()(g, degp, ei3, z2)
    q, h1 = _hop2()(p, h0, deg, ei3, z2)
    out = _fin()(q, h1, deg, b16)
    return out[:N, :C]
